# baseline (speedup 1.0000x reference)
import functools

import jax
import jax.numpy as jnp
from jax import lax
from jax.experimental import pallas as pl
from jax.experimental.pallas import tpu as pltpu
from jax.experimental.pallas import tpu_sc as plsc

F32 = jnp.float32
_NC = 2
_NS = 16
_NW = _NC * _NS
_CH = 72
_NBUF = 4
_NPLANE = 8


def _dims(N, E):
    nch = -(-E // (_NW * _CH))
    nch = -(-nch // _NPLANE) * _NPLANE
    per_w = nch * _CH
    nacc = N + 16
    rps = nacc // _NS
    return per_w, nch, nacc, rps


def _make_deg(N, E):
    W = 128
    per_w, nch, nacc, rps = _dims(N, E)
    ring = max(r for r in (4, 3, 2) if nch % r == 0)
    ngr = nch // ring
    mesh = plsc.VectorSubcoreMesh(core_axis_name="c", subcore_axis_name="s")

    @functools.partial(
        pl.kernel,
        out_type=jax.ShapeDtypeStruct((_NW, rps, W), F32),
        mesh=mesh,
        scratch_types=[
            pltpu.VMEM((nch, _CH), jnp.int32),
            pltpu.VMEM((_CH, W), F32),
            pltpu.VMEM_SHARED((nacc, W), F32),
        ] + [pltpu.SemaphoreType.DMA] * ring,
    )
    def deg_kernel(dst_hbm, ones_hbm, zeros_hbm, out_hbm, idx_v, ones_v,
                   acc_sh, *sems):
        cid = lax.axis_index("c")
        sid = lax.axis_index("s")
        wid = cid * _NS + sid
        pltpu.sync_copy(dst_hbm.at[wid], idx_v)
        pltpu.sync_copy(ones_hbm, ones_v)
        pltpu.sync_copy(zeros_hbm, acc_sh.at[pl.ds(sid * rps, rps)])
        plsc.subcore_barrier()

        for b in range(ring):
            pltpu.async_copy(ones_v, acc_sh.at[idx_v.at[b]], sems[b], add=True)

        def group(g, carry):
            for b in range(ring):
                j = g * ring + b
                pltpu.make_async_copy(ones_v, acc_sh.at[idx_v.at[j]], sems[b]).wait()
                pltpu.async_copy(ones_v, acc_sh.at[idx_v.at[j + ring]], sems[b],
                                 add=True)
            return carry

        lax.fori_loop(0, ngr - 1, group, 0)
        base = (ngr - 1) * ring
        for b in range(ring):
            pltpu.make_async_copy(ones_v, acc_sh.at[idx_v.at[base + b]], sems[b]).wait()
        plsc.subcore_barrier()
        pltpu.sync_copy(acc_sh.at[pl.ds(sid * rps, rps)], out_hbm.at[wid])

    return deg_kernel


def _make_agg(N, D, E):
    per_w, nch, nacc, rps = _dims(N, E)
    P, B = _NPLANE, _NBUF
    ngr = nch // P
    mesh = plsc.VectorSubcoreMesh(core_axis_name="c", subcore_axis_name="s")

    @functools.partial(
        pl.kernel,
        out_type=jax.ShapeDtypeStruct((_NW, rps, D), F32),
        mesh=mesh,
        scratch_types=[pltpu.VMEM((2, _CH), jnp.int32) for _ in range(P)]
        + [pltpu.VMEM((_CH, D), F32) for _ in range(B)]
        + [
            pltpu.VMEM_SHARED((nacc, D), F32),
        ]
        + [pltpu.SemaphoreType.DMA] * (2 * P + B),
    )
    def agg_kernel(x_hbm, eidx_hbm, zeros_hbm, out_hbm, *scr):
        planes = scr[:P]
        bufs = scr[P:P + B]
        acc_sh = scr[P + B]
        isems = scr[P + B + 1:P + B + 1 + P]
        gsems = scr[P + B + 1 + P:P + B + 1 + P + B]
        ssems = scr[P + B + 1 + P + B:]
        cid = lax.axis_index("c")
        sid = lax.axis_index("s")
        wid = cid * _NS + sid

        def load_plane(p, j):
            pltpu.async_copy(eidx_hbm.at[wid, j], planes[p], isems[p])

        def wait_plane(p):
            pltpu.make_async_copy(eidx_hbm.at[wid, 0], planes[p], isems[p]).wait()

        def fire_gather(p, b):
            pltpu.async_copy(x_hbm.at[planes[p].at[0]], bufs[b], gsems[b])

        def wait_gather(b):
            pltpu.make_async_copy(x_hbm.at[planes[0].at[0]], bufs[b], gsems[b]).wait()

        def fire_scatter(b, p):
            pltpu.async_copy(bufs[b], acc_sh.at[planes[p].at[1]], ssems[b], add=True)

        def wait_scatter(b):
            pltpu.make_async_copy(bufs[b], acc_sh.at[planes[0].at[1]], ssems[b]).wait()

        for p in range(P):
            load_plane(p, p)
        pltpu.sync_copy(zeros_hbm, acc_sh.at[pl.ds(sid * rps, rps)])
        plsc.subcore_barrier()

        for b in range(B - 1):
            wait_plane(b)
            fire_gather(b, b)
        wait_gather(0)
        fire_scatter(0, 0)
        wait_plane(3)
        fire_gather(3, 3)

        def group(g, carry):
            for r in range(P):
                i = g * P + r + 1
                b = (r + 1) % B
                p = (r + 1) % P
                bD = r % B
                pD = (r + 4) % P
                wait_gather(b)
                fire_scatter(b, p)
                wait_scatter(bD)
                wait_plane(pD)
                fire_gather(pD, bD)
                load_plane(r, i + 7)
            return carry

        lax.fori_loop(0, (nch - 8) // P, group, 0)
        for i in range(nch - 7, nch):
            b = i % B
            p = i % P
            bD = (i + 3) % B
            pD = (i + 3) % P
            wait_gather(b)
            fire_scatter(b, p)
            wait_scatter(bD)
            if i + 3 <= nch - 1:
                wait_plane(pD)
            fire_gather(pD, bD)

        wait_gather(nch % B)
        wait_gather((nch + 1) % B)
        wait_gather((nch + 2) % B)
        wait_scatter((nch - 1) % B)
        plsc.subcore_barrier()
        pltpu.sync_copy(acc_sh.at[pl.ds(sid * rps, rps)], out_hbm.at[wid])

    return agg_kernel


def _row_block(N):
    return max(r for r in range(8, 513, 8) if N % r == 0)


def _prescale(degp, emb):
    N, D = emb.shape
    R = _row_block(N)

    def body(degp_ref, emb_ref, xs_ref, dis_ref):
        deg = degp_ref[0, :, 0:1] + degp_ref[1, :, 0:1] + 1.0
        dis = lax.rsqrt(deg)
        dis_ref[...] = dis
        xs_ref[...] = emb_ref[...] * dis

    return pl.pallas_call(
        body,
        grid=(N // R,),
        in_specs=[
            pl.BlockSpec((2, R, 128), lambda i: (0, i, 0)),
            pl.BlockSpec((R, D), lambda i: (i, 0)),
        ],
        out_specs=[
            pl.BlockSpec((R, D), lambda i: (i, 0)),
            pl.BlockSpec((R, 1), lambda i: (i, 0)),
        ],
        out_shape=[
            jax.ShapeDtypeStruct((N, D), F32),
            jax.ShapeDtypeStruct((N, 1), F32),
        ],
    )(degp, emb)


def _dense_mid(a, xs, dis, W1, b1, W2):
    N, D = xs.shape
    H = W1.shape[1]
    R = _row_block(N)

    def body(a_ref, xs_ref, dis_ref, W1_ref, b1_ref, W2_ref, out_ref):
        pre = (a_ref[0] + a_ref[1] + xs_ref[...]) * dis_ref[...]
        h1 = jnp.dot(pre, W1_ref[...], preferred_element_type=F32) + b1_ref[...]
        ss = jnp.sum(h1 * h1, axis=1, keepdims=True)
        h = h1 / jnp.maximum(jnp.sqrt(ss), 1e-12)
        x2 = jnp.dot(h, W2_ref[...], preferred_element_type=F32)
        out_ref[...] = x2 * dis_ref[...]

    return pl.pallas_call(
        body,
        grid=(N // R,),
        in_specs=[
            pl.BlockSpec((2, R, D), lambda i: (0, i, 0)),
            pl.BlockSpec((R, D), lambda i: (i, 0)),
            pl.BlockSpec((R, 1), lambda i: (i, 0)),
            pl.BlockSpec((D, H), lambda i: (0, 0)),
            pl.BlockSpec((1, H), lambda i: (0, 0)),
            pl.BlockSpec((H, D), lambda i: (0, 0)),
        ],
        out_specs=pl.BlockSpec((R, D), lambda i: (i, 0)),
        out_shape=jax.ShapeDtypeStruct((N, D), F32),
    )(a, xs, dis, W1, b1, W2)


def _final(q, x2s, dis, b2):
    N, D = x2s.shape
    R = _row_block(N)

    def body(q_ref, x2s_ref, dis_ref, b2_ref, out_ref):
        out_ref[...] = (q_ref[0] + q_ref[1] + x2s_ref[...]) * dis_ref[...] + b2_ref[...]

    return pl.pallas_call(
        body,
        grid=(N // R,),
        in_specs=[
            pl.BlockSpec((2, R, D), lambda i: (0, i, 0)),
            pl.BlockSpec((R, D), lambda i: (i, 0)),
            pl.BlockSpec((R, 1), lambda i: (i, 0)),
            pl.BlockSpec((1, D), lambda i: (0, 0)),
        ],
        out_specs=pl.BlockSpec((R, D), lambda i: (i, 0)),
        out_shape=jax.ShapeDtypeStruct((N, D), F32),
    )(q, x2s, dis, b2)


def kernel(edge_index, emb, W1, b1, W2, b2):
    N, D = emb.shape
    E = edge_index.shape[1]
    per_w, nch, nacc, rps = _dims(N, E)
    pad = _NW * per_w - E

    src = jnp.concatenate([edge_index[0].astype(jnp.int32),
                           jnp.arange(pad, dtype=jnp.int32) % N])
    dst = jnp.concatenate([edge_index[1].astype(jnp.int32),
                           jnp.full((pad,), N, jnp.int32)])
    src3 = src.reshape(_NW, nch, _CH)
    dst3 = dst.reshape(_NW, nch, _CH)
    eidx = jnp.stack([src3, dst3], axis=2)
    ones128 = jnp.ones((_CH, 128), F32)
    zeros128 = jnp.zeros((rps, 128), F32)

    degp = _make_deg(N, E)(dst3, ones128, zeros128).reshape(_NC, nacc, 128)
    xs, dis = _prescale(degp, emb)

    agg_fn = _make_agg(N, D, E)
    a = agg_fn(xs, eidx, zeros128).reshape(_NC, nacc, D)
    x2s = _dense_mid(a, xs, dis, W1, b1.reshape(1, -1), W2)
    q = agg_fn(x2s, eidx, zeros128).reshape(_NC, nacc, D)
    return _final(q, x2s, dis, b2.reshape(1, -1))


# speedup vs baseline: 1.0187x; 1.0187x over previous
import functools

import jax
import jax.numpy as jnp
from jax import lax
from jax.experimental import pallas as pl
from jax.experimental.pallas import tpu as pltpu
from jax.experimental.pallas import tpu_sc as plsc

F32 = jnp.float32
_NC = 2
_NS = 16
_NW = _NC * _NS
_CH = 96
_NBUF = 3
_NPLANE = 6


def _dims(N, E):
    nch = -(-E // (_NW * _CH))
    nch = -(-nch // _NPLANE) * _NPLANE
    per_w = nch * _CH
    nacc = N + 16
    rps = nacc // _NS
    return per_w, nch, nacc, rps


def _make_deg(N, E):
    W = 128
    per_w, nch, nacc, rps = _dims(N, E)
    ring = max(r for r in (4, 3, 2) if nch % r == 0)
    ngr = nch // ring
    mesh = plsc.VectorSubcoreMesh(core_axis_name="c", subcore_axis_name="s")

    @functools.partial(
        pl.kernel,
        out_type=jax.ShapeDtypeStruct((_NW, rps, W), F32),
        mesh=mesh,
        scratch_types=[
            pltpu.VMEM((nch, _CH), jnp.int32),
            pltpu.VMEM((_CH, W), F32),
            pltpu.VMEM_SHARED((nacc, W), F32),
        ] + [pltpu.SemaphoreType.DMA] * ring,
    )
    def deg_kernel(dst_hbm, ones_hbm, zeros_hbm, out_hbm, idx_v, ones_v,
                   acc_sh, *sems):
        cid = lax.axis_index("c")
        sid = lax.axis_index("s")
        wid = cid * _NS + sid
        pltpu.sync_copy(dst_hbm.at[wid], idx_v)
        pltpu.sync_copy(ones_hbm, ones_v)
        pltpu.sync_copy(zeros_hbm, acc_sh.at[pl.ds(sid * rps, rps)])
        plsc.subcore_barrier()

        for b in range(ring):
            pltpu.async_copy(ones_v, acc_sh.at[idx_v.at[b]], sems[b], add=True)

        def group(g, carry):
            for b in range(ring):
                j = g * ring + b
                pltpu.make_async_copy(ones_v, acc_sh.at[idx_v.at[j]], sems[b]).wait()
                pltpu.async_copy(ones_v, acc_sh.at[idx_v.at[j + ring]], sems[b],
                                 add=True)
            return carry

        lax.fori_loop(0, ngr - 1, group, 0)
        base = (ngr - 1) * ring
        for b in range(ring):
            pltpu.make_async_copy(ones_v, acc_sh.at[idx_v.at[base + b]], sems[b]).wait()
        plsc.subcore_barrier()
        pltpu.sync_copy(acc_sh.at[pl.ds(sid * rps, rps)], out_hbm.at[wid])

    return deg_kernel


def _make_agg(N, D, E):
    per_w, nch, nacc, rps = _dims(N, E)
    P, B = _NPLANE, _NBUF
    ngr = nch // P
    mesh = plsc.VectorSubcoreMesh(core_axis_name="c", subcore_axis_name="s")

    @functools.partial(
        pl.kernel,
        out_type=jax.ShapeDtypeStruct((_NW, rps, D), F32),
        mesh=mesh,
        scratch_types=[pltpu.VMEM((2, _CH), jnp.int32) for _ in range(P)]
        + [pltpu.VMEM((_CH, D), F32) for _ in range(B)]
        + [
            pltpu.VMEM_SHARED((nacc, D), F32),
        ]
        + [pltpu.SemaphoreType.DMA] * (2 * P + B),
    )
    def agg_kernel(x_hbm, eidx_hbm, zeros_hbm, out_hbm, *scr):
        planes = scr[:P]
        bufs = scr[P:P + B]
        acc_sh = scr[P + B]
        isems = scr[P + B + 1:P + B + 1 + P]
        gsems = scr[P + B + 1 + P:P + B + 1 + P + B]
        ssems = scr[P + B + 1 + P + B:]
        cid = lax.axis_index("c")
        sid = lax.axis_index("s")
        wid = cid * _NS + sid

        def load_plane(p, j):
            pltpu.async_copy(eidx_hbm.at[wid, j], planes[p], isems[p])

        def wait_plane(p):
            pltpu.make_async_copy(eidx_hbm.at[wid, 0], planes[p], isems[p]).wait()

        def fire_gather(p, b):
            pltpu.async_copy(x_hbm.at[planes[p].at[0]], bufs[b], gsems[b])

        def wait_gather(b):
            pltpu.make_async_copy(x_hbm.at[planes[0].at[0]], bufs[b], gsems[b]).wait()

        def fire_scatter(b, p):
            pltpu.async_copy(bufs[b], acc_sh.at[planes[p].at[1]], ssems[b], add=True)

        def wait_scatter(b):
            pltpu.make_async_copy(bufs[b], acc_sh.at[planes[0].at[1]], ssems[b]).wait()

        for p in range(P):
            load_plane(p, p)
        pltpu.sync_copy(zeros_hbm, acc_sh.at[pl.ds(sid * rps, rps)])
        plsc.subcore_barrier()

        wait_plane(0)
        fire_gather(0, 0)
        wait_plane(1)
        fire_gather(1, 1)
        wait_gather(0)
        fire_scatter(0, 0)
        wait_plane(2)
        fire_gather(2, 2)

        def group(g, carry):
            for r in range(P):
                i = g * P + r + 1
                b = (r + 1) % B
                p = (r + 1) % P
                bD = r % B
                pD = (r + 3) % P
                wait_gather(b)
                fire_scatter(b, p)
                wait_scatter(bD)
                wait_plane(pD)
                fire_gather(pD, bD)
                load_plane(r, i + 5)
            return carry

        lax.fori_loop(0, (nch - 6) // P, group, 0)
        for i in range(nch - 5, nch):
            b = i % B
            p = i % P
            bD = (i + 2) % B
            pD = (i + 2) % P
            wait_gather(b)
            fire_scatter(b, p)
            wait_scatter(bD)
            if i + 2 <= nch - 1:
                wait_plane(pD)
            fire_gather(pD, bD)

        wait_gather(nch % B)
        wait_gather((nch + 1) % B)
        wait_scatter((nch - 1) % B)
        plsc.subcore_barrier()
        pltpu.sync_copy(acc_sh.at[pl.ds(sid * rps, rps)], out_hbm.at[wid])

    return agg_kernel


def _row_block(N):
    return max(r for r in range(8, 513, 8) if N % r == 0)


def _prescale(degp, emb):
    N, D = emb.shape
    R = _row_block(N)

    def body(degp_ref, emb_ref, xs_ref, dis_ref):
        deg = degp_ref[0, :, 0:1] + degp_ref[1, :, 0:1] + 1.0
        dis = lax.rsqrt(deg)
        dis_ref[...] = dis
        xs_ref[...] = emb_ref[...] * dis

    return pl.pallas_call(
        body,
        grid=(N // R,),
        in_specs=[
            pl.BlockSpec((2, R, 128), lambda i: (0, i, 0)),
            pl.BlockSpec((R, D), lambda i: (i, 0)),
        ],
        out_specs=[
            pl.BlockSpec((R, D), lambda i: (i, 0)),
            pl.BlockSpec((R, 1), lambda i: (i, 0)),
        ],
        out_shape=[
            jax.ShapeDtypeStruct((N, D), F32),
            jax.ShapeDtypeStruct((N, 1), F32),
        ],
    )(degp, emb)


def _dense_mid(a, xs, dis, W1, b1, W2):
    N, D = xs.shape
    H = W1.shape[1]
    R = _row_block(N)

    def body(a_ref, xs_ref, dis_ref, W1_ref, b1_ref, W2_ref, out_ref):
        pre = (a_ref[0] + a_ref[1] + xs_ref[...]) * dis_ref[...]
        h1 = jnp.dot(pre, W1_ref[...], preferred_element_type=F32) + b1_ref[...]
        ss = jnp.sum(h1 * h1, axis=1, keepdims=True)
        h = h1 / jnp.maximum(jnp.sqrt(ss), 1e-12)
        x2 = jnp.dot(h, W2_ref[...], preferred_element_type=F32)
        out_ref[...] = x2 * dis_ref[...]

    return pl.pallas_call(
        body,
        grid=(N // R,),
        in_specs=[
            pl.BlockSpec((2, R, D), lambda i: (0, i, 0)),
            pl.BlockSpec((R, D), lambda i: (i, 0)),
            pl.BlockSpec((R, 1), lambda i: (i, 0)),
            pl.BlockSpec((D, H), lambda i: (0, 0)),
            pl.BlockSpec((1, H), lambda i: (0, 0)),
            pl.BlockSpec((H, D), lambda i: (0, 0)),
        ],
        out_specs=pl.BlockSpec((R, D), lambda i: (i, 0)),
        out_shape=jax.ShapeDtypeStruct((N, D), F32),
    )(a, xs, dis, W1, b1, W2)


def _final(q, x2s, dis, b2):
    N, D = x2s.shape
    R = _row_block(N)

    def body(q_ref, x2s_ref, dis_ref, b2_ref, out_ref):
        out_ref[...] = (q_ref[0] + q_ref[1] + x2s_ref[...]) * dis_ref[...] + b2_ref[...]

    return pl.pallas_call(
        body,
        grid=(N // R,),
        in_specs=[
            pl.BlockSpec((2, R, D), lambda i: (0, i, 0)),
            pl.BlockSpec((R, D), lambda i: (i, 0)),
            pl.BlockSpec((R, 1), lambda i: (i, 0)),
            pl.BlockSpec((1, D), lambda i: (0, 0)),
        ],
        out_specs=pl.BlockSpec((R, D), lambda i: (i, 0)),
        out_shape=jax.ShapeDtypeStruct((N, D), F32),
    )(q, x2s, dis, b2)


def kernel(edge_index, emb, W1, b1, W2, b2):
    N, D = emb.shape
    E = edge_index.shape[1]
    per_w, nch, nacc, rps = _dims(N, E)
    pad = _NW * per_w - E

    src = jnp.concatenate([edge_index[0].astype(jnp.int32),
                           jnp.arange(pad, dtype=jnp.int32) % N])
    dst = jnp.concatenate([edge_index[1].astype(jnp.int32),
                           jnp.full((pad,), N, jnp.int32)])
    src3 = src.reshape(_NW, nch, _CH)
    dst3 = dst.reshape(_NW, nch, _CH)
    eidx = jnp.stack([src3, dst3], axis=2)
    ones128 = jnp.ones((_CH, 128), F32)
    zeros128 = jnp.zeros((rps, 128), F32)

    degp = _make_deg(N, E)(dst3, ones128, zeros128).reshape(_NC, nacc, 128)
    xs, dis = _prescale(degp, emb)

    agg_fn = _make_agg(N, D, E)
    a = agg_fn(xs, eidx, zeros128).reshape(_NC, nacc, D)
    x2s = _dense_mid(a, xs, dis, W1, b1.reshape(1, -1), W2)
    q = agg_fn(x2s, eidx, zeros128).reshape(_NC, nacc, D)
    return _final(q, x2s, dis, b2.reshape(1, -1))
